# baseline (speedup 1.0000x reference)
import functools

import jax
import jax.numpy as jnp
from jax import lax
from jax.experimental import pallas as pl
from jax.experimental.pallas import tpu as pltpu
from jax.experimental.pallas import tpu_sc as plsc

F = 26
V = 100000
D = 16
B = 4096
NC = 2
NS = 16
NW = NC * NS
B_PER_W = B // NW
G = 4
GB = G * F * 4
NST = 4
CH = GB // NST
IDXW = 512
NG = B_PER_W // G
LIN_W = 32
GLIN = G * LIN_W


def _ffm_body(idx_hbm, lidx_hbm, emb_hbm, lin_hbm, bias_hbm, out_hbm,
              idx_v, rows_v, lidx_v, lin_v, bias_v, out_v, sem0, sem1):
    wid = lax.axis_index("s") * NC + lax.axis_index("c")
    gbase = wid * NG
    pltpu.sync_copy(bias_hbm, bias_v)
    bias_s = bias_v[pl.ds(0, 16)][0]
    lanes = lax.iota(jnp.int32, 16)
    sems = (sem0, sem1)

    def stage(g, slot):
        grow = gbase + g
        pltpu.sync_copy(idx_hbm.at[grow], idx_v.at[pl.ds(slot * IDXW, IDXW)])
        pltpu.sync_copy(lidx_hbm.at[grow], lidx_v.at[pl.ds(slot * GLIN, GLIN)])
        for s in range(NST):
            pltpu.async_copy(emb_hbm.at[idx_v.at[pl.ds(slot * IDXW + s * CH, CH)]],
                             rows_v.at[pl.ds(slot * GB + s * CH, CH)],
                             sems[slot])
        pltpu.async_copy(lin_hbm.at[lidx_v.at[pl.ds(slot * GLIN, GLIN)]],
                         lin_v.at[pl.ds(slot * GLIN, GLIN)], sems[slot])

    def wait_slot(slot):
        pltpu.make_async_copy(emb_hbm.at[pl.ds(0, GB)],
                              rows_v.at[pl.ds(slot * GB, GB)],
                              sems[slot]).wait()
        pltpu.make_async_copy(lin_hbm.at[pl.ds(0, GLIN)],
                              lin_v.at[pl.ds(slot * GLIN, GLIN)],
                              sems[slot]).wait()

    def compute(g, slot_off):
        def row_body(r, carry):
            rbase = slot_off * GB + r * (F * 4)
            lbase = slot_off * GLIN + r * LIN_W
            accs = [jnp.zeros((D,), jnp.float32) for _ in range(4)]
            k = 0
            for i in range(F):
                for j in range(i + 1, F):
                    a = rows_v[rbase + i * 4 + j // 8, pl.ds((j % 8) * D, D)]
                    bb = rows_v[rbase + j * 4 + i // 8, pl.ds((i % 8) * D, D)]
                    accs[k % 4] = accs[k % 4] + a * bb
                    k += 1
            tv = (accs[0] + accs[1] + accs[2] + accs[3]
                  + lin_v[pl.ds(lbase, 16)] + lin_v[pl.ds(lbase + 16, 16)])
            for sh in (8, 4, 2, 1):
                tv = tv + tv.at[lanes ^ sh].get(mode="promise_in_bounds")
            tv = tv + bias_s
            bi = g * G + r
            off16 = (bi // 16) * 16
            cur = out_v[pl.ds(off16, 16)]
            out_v[pl.ds(off16, 16)] = jnp.where(lanes == bi % 16, tv, cur)
            return carry

        lax.fori_loop(0, G, row_body, 0)

    stage(0, 0)

    def body(g, carry):
        even = lax.rem(g, 2) == 0

        @pl.when(even)
        def _():
            stage(g + 1, 1)

        @pl.when(jnp.logical_and(jnp.logical_not(even), g < NG - 1))
        def _():
            stage(g + 1, 0)

        @pl.when(even)
        def _():
            wait_slot(0)

        @pl.when(jnp.logical_not(even))
        def _():
            wait_slot(1)

        compute(g, lax.rem(g, 2))
        return carry

    lax.fori_loop(0, NG, body, 0)

    for k in range(B_PER_W // 16):
        v = out_v[pl.ds(k * 16, 16)]
        out_v[pl.ds(k * 16, 16)] = 1.0 / (1.0 + jnp.exp(-v))
    pltpu.sync_copy(out_v, out_hbm.at[pl.ds(wid * B_PER_W, B_PER_W)])


RBV = 400


def _repack_body(in_ref, out_ref):
    t = jnp.swapaxes(in_ref[...], 0, 1).reshape(RBV, F * D)
    t = jnp.pad(t, ((0, 0), (0, 512 - F * D)))
    out_ref[...] = t.reshape(4 * RBV, 128)


@jax.jit
def _ffm_sc(x, emb_tables, linear_w, bias):
    emb2 = pl.pallas_call(
        _repack_body,
        grid=(V // RBV,),
        in_specs=[pl.BlockSpec((F, RBV, D), lambda i: (0, i, 0))],
        out_specs=pl.BlockSpec((4 * RBV, 128), lambda i: (i, 0)),
        out_shape=jax.ShapeDtypeStruct((4 * V, 128), jnp.float32),
    )(emb_tables)
    xi = x.astype(jnp.int32)
    idx = (xi[:, :, None] * 4 + jnp.arange(4, dtype=jnp.int32)).reshape(B // G, GB)
    idx = jnp.pad(idx, ((0, 0), (0, IDXW - GB)))
    lidx = jnp.pad(xi, ((0, 0), (0, LIN_W - F)), constant_values=V)
    lidx = lidx.reshape(B // G, GLIN)
    lin_pad = jnp.concatenate([linear_w.reshape(V),
                               jnp.zeros((1,), jnp.float32)])
    bias16 = jnp.pad(bias.astype(jnp.float32).reshape(1), (0, 15))
    run = pl.kernel(
        _ffm_body,
        out_type=jax.ShapeDtypeStruct((B,), jnp.float32),
        mesh=plsc.VectorSubcoreMesh(core_axis_name="c", subcore_axis_name="s"),
        scratch_types=[
            pltpu.VMEM((2 * IDXW,), jnp.int32),
            pltpu.VMEM((2 * GB, 128), jnp.float32),
            pltpu.VMEM((2 * GLIN,), jnp.int32),
            pltpu.VMEM((2 * GLIN,), jnp.float32),
            pltpu.VMEM((16,), jnp.float32),
            pltpu.VMEM((B_PER_W,), jnp.float32),
            pltpu.SemaphoreType.DMA,
            pltpu.SemaphoreType.DMA,
        ],
    )
    return run(idx, lidx, emb2, lin_pad, bias16)


def kernel(x, emb_tables, linear_w, bias):
    out = _ffm_sc(x, emb_tables, linear_w, bias)
    return out.reshape(B, 1)


# speedup vs baseline: 1.3927x; 1.3927x over previous
import functools

import jax
import jax.numpy as jnp
from jax import lax
from jax.experimental import pallas as pl
from jax.experimental.pallas import tpu as pltpu
from jax.experimental.pallas import tpu_sc as plsc

F = 26
V = 100000
D = 16
B = 4096
BW = 512
NC = 2
NS = 16
NW = NC * NS
B_PER_W = B // NW
G = 4
GB = G * F
IDXW = 128
NG = B_PER_W // G
LIN_W = 32
GLIN = G * LIN_W


def _ffm_body(idx_hbm, lidx_hbm, emb_hbm, lin_hbm, bias_hbm, out_hbm,
              idx_v, rows_v, lidx_v, lin_v, bias_v, out_v, sem0, sem1):
    wid = lax.axis_index("s") * NC + lax.axis_index("c")
    gbase = wid * NG
    pltpu.sync_copy(bias_hbm, bias_v)
    bias_s = bias_v[pl.ds(0, 16)][0]
    lanes = lax.iota(jnp.int32, 16)
    sems = (sem0, sem1)

    def stage(g, slot):
        grow = gbase + g
        pltpu.sync_copy(idx_hbm.at[grow], idx_v.at[pl.ds(slot * IDXW, IDXW)])
        pltpu.sync_copy(lidx_hbm.at[grow], lidx_v.at[pl.ds(slot * GLIN, GLIN)])
        pltpu.async_copy(emb_hbm.at[idx_v.at[pl.ds(slot * IDXW, GB)]],
                         rows_v.at[pl.ds(slot * GB, GB)], sems[slot])
        pltpu.async_copy(lin_hbm.at[lidx_v.at[pl.ds(slot * GLIN, GLIN)]],
                         lin_v.at[pl.ds(slot * GLIN, GLIN)], sems[slot])

    def wait_slot(slot):
        pltpu.make_async_copy(emb_hbm.at[pl.ds(0, GB)],
                              rows_v.at[pl.ds(slot * GB, GB)],
                              sems[slot]).wait()
        pltpu.make_async_copy(lin_hbm.at[pl.ds(0, GLIN)],
                              lin_v.at[pl.ds(slot * GLIN, GLIN)],
                              sems[slot]).wait()

    def compute(g, slot_off):
        def row_body(r, carry):
            rbase = slot_off * GB + r * F
            lbase = slot_off * GLIN + r * LIN_W
            accs = [jnp.zeros((D,), jnp.float32) for _ in range(4)]
            k = 0
            for i in range(F):
                for j in range(i + 1, F):
                    a = rows_v[rbase + i, pl.ds(j * D, D)]
                    bb = rows_v[rbase + j, pl.ds(i * D, D)]
                    accs[k % 4] = accs[k % 4] + a * bb
                    k += 1
            tv = (accs[0] + accs[1] + accs[2] + accs[3]
                  + lin_v[pl.ds(lbase, 16)] + lin_v[pl.ds(lbase + 16, 16)])
            for sh in (8, 4, 2, 1):
                tv = tv + tv.at[lanes ^ sh].get(mode="promise_in_bounds")
            tv = tv + bias_s
            bi = g * G + r
            off16 = (bi // 16) * 16
            cur = out_v[pl.ds(off16, 16)]
            out_v[pl.ds(off16, 16)] = jnp.where(lanes == bi % 16, tv, cur)
            return carry

        lax.fori_loop(0, G, row_body, 0)

    stage(0, 0)

    def body(g, carry):
        even = lax.rem(g, 2) == 0

        @pl.when(even)
        def _():
            stage(g + 1, 1)

        @pl.when(jnp.logical_and(jnp.logical_not(even), g < NG - 1))
        def _():
            stage(g + 1, 0)

        @pl.when(even)
        def _():
            wait_slot(0)

        @pl.when(jnp.logical_not(even))
        def _():
            wait_slot(1)

        compute(g, lax.rem(g, 2))
        return carry

    lax.fori_loop(0, NG, body, 0)

    for k in range(B_PER_W // 16):
        v = out_v[pl.ds(k * 16, 16)]
        out_v[pl.ds(k * 16, 16)] = 1.0 / (1.0 + jnp.exp(-v))
    pltpu.sync_copy(out_v, out_hbm.at[pl.ds(wid * B_PER_W, B_PER_W)])


@jax.jit
def _ffm_sc(x, emb_tables, linear_w, bias):
    emb2 = emb_tables.transpose(1, 0, 2).reshape(V, F * D)
    emb2 = jnp.pad(emb2, ((0, 0), (0, BW - F * D)))
    xi = x.astype(jnp.int32)
    idx = xi.reshape(B // G, GB)
    idx = jnp.pad(idx, ((0, 0), (0, IDXW - GB)))
    lidx = jnp.pad(xi, ((0, 0), (0, LIN_W - F)), constant_values=V)
    lidx = lidx.reshape(B // G, GLIN)
    lin_pad = jnp.concatenate([linear_w.reshape(V),
                               jnp.zeros((1,), jnp.float32)])
    bias16 = jnp.pad(bias.astype(jnp.float32).reshape(1), (0, 15))
    run = pl.kernel(
        _ffm_body,
        out_type=jax.ShapeDtypeStruct((B,), jnp.float32),
        mesh=plsc.VectorSubcoreMesh(core_axis_name="c", subcore_axis_name="s"),
        scratch_types=[
            pltpu.VMEM((2 * IDXW,), jnp.int32),
            pltpu.VMEM((2 * GB, BW), jnp.float32),
            pltpu.VMEM((2 * GLIN,), jnp.int32),
            pltpu.VMEM((2 * GLIN,), jnp.float32),
            pltpu.VMEM((16,), jnp.float32),
            pltpu.VMEM((B_PER_W,), jnp.float32),
            pltpu.SemaphoreType.DMA,
            pltpu.SemaphoreType.DMA,
        ],
    )
    return run(idx, lidx, emb2, lin_pad, bias16)


def kernel(x, emb_tables, linear_w, bias):
    out = _ffm_sc(x, emb_tables, linear_w, bias)
    return out.reshape(B, 1)
